# baseline (speedup 1.0000x reference)
import jax
import jax.numpy as jnp
from jax import lax
from jax.experimental import pallas as pl
from jax.experimental.pallas import tpu as pltpu
from jax.experimental.pallas import tpu_sc as plsc

_B = 16384
_EMB = 64
_L = 16
_NW = 32
_BW = _B // _NW
_CH = 128
_NCH = _BW // _CH
_RB = 1024


def _proj_body(x_ref, w_ref, o_ref):
    x = x_ref[...]
    w = w_ref[...]
    o_ref[...] = jnp.sum(x.reshape(_RB // 128, 128, _EMB)
                         * w.reshape(1, 1, _EMB), axis=-1)


def _project(table, w2d, grid):
    return pl.pallas_call(
        _proj_body,
        out_shape=jax.ShapeDtypeStruct((grid * (_RB // 128), 128),
                                       jnp.float32),
        grid=(grid,),
        in_specs=[pl.BlockSpec((_RB, _EMB), lambda i: (i, 0)),
                  pl.BlockSpec((1, _EMB), lambda i: (0, 0))],
        out_specs=pl.BlockSpec((_RB // 128, 128), lambda i: (i, 0)),
    )(table, w2d)


def _gat_body(users_ref, items_ref, uproj, iproj, bref, out_ref,
              ubuf, ibuf, urow, irow, ug0, ug1, ig0, ig1, outb, bbuf,
              sem0, sem1):
    wid = lax.axis_index("s") * 2 + lax.axis_index("c")
    base = wid * _BW

    pltpu.sync_copy(users_ref.at[pl.ds(base, _BW)], ubuf)
    pltpu.sync_copy(items_ref.at[pl.ds(base, _BW)], ibuf)
    pltpu.sync_copy(bref, bbuf)

    def rows_body(t, carry):
        urow[pl.ds(t * _L, _L)] = ubuf[pl.ds(t * _L, _L)] >> 7
        irow[pl.ds(t * _L, _L)] = ibuf[pl.ds(t * _L, _L)] >> 7
        return carry

    lax.fori_loop(0, _BW // _L, rows_body, 0)

    bv = bbuf[...]
    iota = lax.iota(jnp.int32, _L)
    ugs = [ug0, ug1]
    igs = [ig0, ig1]
    sems = [sem0, sem1]

    def fire(c):
        s = sems[c % 2]
        du = pltpu.async_copy(uproj.at[urow.at[pl.ds(c * _CH, _CH)]],
                              ugs[c % 2], s)
        di = pltpu.async_copy(iproj.at[irow.at[pl.ds(c * _CH, _CH)]],
                              igs[c % 2], s)
        return du, di

    pending = fire(0)
    for c in range(_NCH):
        nxt = fire(c + 1) if c + 1 < _NCH else None
        du, di = pending
        du.wait()
        di.wait()
        ug = ugs[c % 2]
        ig = igs[c % 2]

        def group(g, carry):
            rb = g * _L
            rows16 = iota + rb
            uv = ubuf[pl.ds(c * _CH + rb, _L)]
            iv = ibuf[pl.ds(c * _CH + rb, _L)]
            uvals = plsc.load_gather(ug, [rows16, uv & 127])
            ivals = plsc.load_gather(ig, [rows16, iv & 127])
            outb[pl.ds(c * _CH + rb, _L)] = uvals + ivals + bv
            return carry

        lax.fori_loop(0, _CH // _L, group, 0)
        pending = nxt

    pltpu.sync_copy(outb, out_ref.at[pl.ds(base, _BW)])


def kernel(users, items, user_table, item_table, W, b):
    users1d = users.astype(jnp.int32)
    items1d = items.astype(jnp.int32)
    wflat = W.reshape(2 * _EMB)
    wu = wflat[:_EMB].reshape(1, _EMB)
    wi = wflat[_EMB:].reshape(1, _EMB)
    b16 = jnp.broadcast_to(b, (_L,))

    gu = -(-user_table.shape[0] // _RB)
    gi = -(-item_table.shape[0] // _RB)
    uproj = _project(user_table, wu, gu)
    iproj = _project(item_table, wi, gi)

    mesh = plsc.VectorSubcoreMesh(core_axis_name="c", subcore_axis_name="s")
    f = pl.kernel(
        _gat_body,
        out_type=jax.ShapeDtypeStruct((_B,), jnp.float32),
        mesh=mesh,
        compiler_params=pltpu.CompilerParams(
            needs_layout_passes=False, use_tc_tiling_on_sc=True),
        scratch_types=[
            pltpu.VMEM((_BW,), jnp.int32),
            pltpu.VMEM((_BW,), jnp.int32),
            pltpu.VMEM((_BW,), jnp.int32),
            pltpu.VMEM((_BW,), jnp.int32),
            pltpu.VMEM((_CH, 128), jnp.float32),
            pltpu.VMEM((_CH, 128), jnp.float32),
            pltpu.VMEM((_CH, 128), jnp.float32),
            pltpu.VMEM((_CH, 128), jnp.float32),
            pltpu.VMEM((_BW,), jnp.float32),
            pltpu.VMEM((_L,), jnp.float32),
            pltpu.SemaphoreType.DMA,
            pltpu.SemaphoreType.DMA,
        ],
    )
    out = f(users1d, items1d, uproj, iproj, b16)
    return out.reshape(_B, 1)


# speedup vs baseline: 1.5383x; 1.5383x over previous
import jax
import jax.numpy as jnp
from jax import lax
from jax.experimental import pallas as pl
from jax.experimental.pallas import tpu as pltpu
from jax.experimental.pallas import tpu_sc as plsc

_B = 16384
_EMB = 64
_L = 16
_NW = 32
_BW = _B // _NW
_CH = 128
_NCH = _BW // _CH
_RB = 4096


def _proj_body(x_ref, w_ref, o_ref):
    x = x_ref[...]
    w = w_ref[...]
    o_ref[...] = jnp.sum(x.reshape(_RB // 128, 128, _EMB)
                         * w.reshape(1, 1, _EMB), axis=-1)


def _project(table, w2d, grid):
    return pl.pallas_call(
        _proj_body,
        out_shape=jax.ShapeDtypeStruct((grid * (_RB // 128), 128),
                                       jnp.float32),
        grid=(grid,),
        in_specs=[pl.BlockSpec((_RB, _EMB), lambda i: (i, 0)),
                  pl.BlockSpec((1, _EMB), lambda i: (0, 0))],
        out_specs=pl.BlockSpec((_RB // 128, 128), lambda i: (i, 0)),
    )(table, w2d)


def _gat_body(users_ref, items_ref, uproj, iproj, bref, out_ref,
              ubuf, ibuf, urow, irow, ug0, ug1, ig0, ig1, outb, bbuf,
              sem0, sem1):
    wid = lax.axis_index("s") * 2 + lax.axis_index("c")
    base = wid * _BW

    pltpu.sync_copy(users_ref.at[pl.ds(base, _BW)], ubuf)
    pltpu.sync_copy(items_ref.at[pl.ds(base, _BW)], ibuf)
    pltpu.sync_copy(bref, bbuf)

    def rows_body(t, carry):
        urow[pl.ds(t * _L, _L)] = ubuf[pl.ds(t * _L, _L)] >> 7
        irow[pl.ds(t * _L, _L)] = ibuf[pl.ds(t * _L, _L)] >> 7
        return carry

    lax.fori_loop(0, _BW // _L, rows_body, 0)

    bv = bbuf[...]
    iota = lax.iota(jnp.int32, _L)
    ugs = [ug0, ug1]
    igs = [ig0, ig1]
    sems = [sem0, sem1]

    def fire(c):
        s = sems[c % 2]
        du = pltpu.async_copy(uproj.at[urow.at[pl.ds(c * _CH, _CH)]],
                              ugs[c % 2], s)
        di = pltpu.async_copy(iproj.at[irow.at[pl.ds(c * _CH, _CH)]],
                              igs[c % 2], s)
        return du, di

    pending = fire(0)
    for c in range(_NCH):
        nxt = fire(c + 1) if c + 1 < _NCH else None
        du, di = pending
        du.wait()
        di.wait()
        ug = ugs[c % 2]
        ig = igs[c % 2]

        def group(g, carry):
            rb = g * _L
            rows16 = iota + rb
            uv = ubuf[pl.ds(c * _CH + rb, _L)]
            iv = ibuf[pl.ds(c * _CH + rb, _L)]
            uvals = plsc.load_gather(ug, [rows16, uv & 127])
            ivals = plsc.load_gather(ig, [rows16, iv & 127])
            outb[pl.ds(c * _CH + rb, _L)] = uvals + ivals + bv
            return carry

        lax.fori_loop(0, _CH // _L, group, 0)
        pending = nxt

    pltpu.sync_copy(outb, out_ref.at[pl.ds(base, _BW)])


def kernel(users, items, user_table, item_table, W, b):
    users1d = users.astype(jnp.int32)
    items1d = items.astype(jnp.int32)
    wflat = W.reshape(2 * _EMB)
    wu = wflat[:_EMB].reshape(1, _EMB)
    wi = wflat[_EMB:].reshape(1, _EMB)
    b16 = jnp.broadcast_to(b, (_L,))

    gu = -(-user_table.shape[0] // _RB)
    gi = -(-item_table.shape[0] // _RB)
    uproj = _project(user_table, wu, gu)
    iproj = _project(item_table, wi, gi)

    mesh = plsc.VectorSubcoreMesh(core_axis_name="c", subcore_axis_name="s")
    f = pl.kernel(
        _gat_body,
        out_type=jax.ShapeDtypeStruct((_B,), jnp.float32),
        mesh=mesh,
        compiler_params=pltpu.CompilerParams(
            needs_layout_passes=False, use_tc_tiling_on_sc=True),
        scratch_types=[
            pltpu.VMEM((_BW,), jnp.int32),
            pltpu.VMEM((_BW,), jnp.int32),
            pltpu.VMEM((_BW,), jnp.int32),
            pltpu.VMEM((_BW,), jnp.int32),
            pltpu.VMEM((_CH, 128), jnp.float32),
            pltpu.VMEM((_CH, 128), jnp.float32),
            pltpu.VMEM((_CH, 128), jnp.float32),
            pltpu.VMEM((_CH, 128), jnp.float32),
            pltpu.VMEM((_BW,), jnp.float32),
            pltpu.VMEM((_L,), jnp.float32),
            pltpu.SemaphoreType.DMA,
            pltpu.SemaphoreType.DMA,
        ],
    )
    out = f(users1d, items1d, uproj, iproj, b16)
    return out.reshape(_B, 1)
